# Initial kernel scaffold; baseline (speedup 1.0000x reference)
import jax
import jax.numpy as jnp
from jax.experimental import pallas as pl


def kernel(x, table, bias):
    raise NotImplementedError("write your pallas kernel here")


# speedup vs baseline: 1.0516x; 1.0516x over previous
import functools

import jax
import jax.numpy as jnp
from jax import lax
from jax.experimental import pallas as pl
from jax.experimental.pallas import tpu as pltpu
from jax.experimental.pallas import tpu_sc as plsc

_NUM_FIELDS = 26
_FIELD_DIM = 100000
_BATCH = 4096
_LANES = 16
_NC = 2
_NS = 16
_NW = _NC * _NS
_BPW = _BATCH // _NW


def _make_sc_kernel():
    mesh = plsc.VectorSubcoreMesh(core_axis_name="c", subcore_axis_name="s")

    @functools.partial(
        pl.kernel,
        mesh=mesh,
        out_type=jax.ShapeDtypeStruct((_BATCH,), jnp.float32),
        scratch_types=[
            pltpu.VMEM((_NUM_FIELDS, _BPW), jnp.int32),
            pltpu.VMEM((_NUM_FIELDS * _BPW,), jnp.int32),
            pltpu.VMEM((_NUM_FIELDS * _BPW,), jnp.float32),
            pltpu.VMEM((_LANES,), jnp.float32),
            pltpu.VMEM((_BPW,), jnp.float32),
            pltpu.SemaphoreType.DMA,
        ],
    )
    def k(xt_hbm, table_hbm, bias_hbm, out_hbm,
          xb_v, idx_v, rows_v, bias_v, ob_v, sem):
        wid = lax.axis_index("s") * _NC + lax.axis_index("c")
        base = wid * _BPW
        pltpu.sync_copy(xt_hbm.at[wid], xb_v)
        pltpu.sync_copy(bias_hbm, bias_v)
        nv = _BPW // _LANES
        for f in range(_NUM_FIELDS):
            off = f * _FIELD_DIM
            for c in range(nv):
                sl = pl.ds(c * _LANES, _LANES)
                idx_v[pl.ds(f * _BPW + c * _LANES, _LANES)] = xb_v[f, sl] + off
        pltpu.async_copy(table_hbm.at[idx_v], rows_v, sem).wait()
        for c in range(nv):
            acc = bias_v[...]
            for f in range(_NUM_FIELDS):
                acc = acc + rows_v[pl.ds(f * _BPW + c * _LANES, _LANES)]
            ob_v[pl.ds(c * _LANES, _LANES)] = 1.0 / (1.0 + jnp.exp(-acc))
        pltpu.sync_copy(ob_v, out_hbm.at[pl.ds(base, _BPW)])

    return k


_sc_kernel = _make_sc_kernel()


@jax.jit
def kernel(x, table, bias):
    xt = x.T.reshape(_NUM_FIELDS, _NW, _BPW).transpose(1, 0, 2)
    table_flat = table.reshape(-1)
    bias16 = jnp.broadcast_to(bias, (_LANES,))
    return _sc_kernel(xt, table_flat, bias16)
